# Initial kernel scaffold; baseline (speedup 1.0000x reference)
import functools

import jax
import jax.numpy as jnp
from jax import lax
from jax.experimental import pallas as pl
from jax.experimental.pallas import tpu as pltpu
from jax.experimental.pallas import tpu_sc as plsc

_N = 10000
_E = 320000
_FIN = 128
_H = 256
_HALF = 128
_C = 10
_EPS = 1e-5

_NC = 2
_NS = 16
_CH = 128

_ROWS_PT = _N // _NS

_mesh = plsc.VectorSubcoreMesh(
    core_axis_name="c", subcore_axis_name="s", num_cores=_NC, num_subcores=_NS
)


def _zero_vmem_rows(buf, nrows, width):
    def body(i, _):
        for j in range(width // 16):
            buf[i, pl.ds(j * 16, 16)] = jnp.zeros((16,), jnp.float32)
        return 0

    lax.fori_loop(0, nrows, body, 0)


def _fill_ones_rows(buf, nrows, width):
    def body(i, _):
        for j in range(width // 16):
            buf[i, pl.ds(j * 16, 16)] = jnp.ones((16,), jnp.float32)
        return 0

    lax.fori_loop(0, nrows, body, 0)


def _zero_my_shared_slice(acc_sh, zbuf, sid, width):
    base = sid * _ROWS_PT
    for off, size in ((0, 128), (128, 128), (256, 128), (384, 128), (512, 113)):
        pltpu.sync_copy(zbuf.at[pl.ds(0, size)], acc_sh.at[pl.ds(base + off, size)])


_DEG_EPT = _E // (_NC * _NS)
_DEG_FULL = _DEG_EPT // _CH
_DEG_REM = _DEG_EPT - _DEG_FULL * _CH


def _deg_body(dst_hbm, out_hbm, deg_sh, upd, upd_r, dst_buf, dst_r, sem):
    c = lax.axis_index("c")
    sid = lax.axis_index("s")

    _zero_vmem_rows(upd, _CH, 16)
    _zero_my_shared_slice(deg_sh, upd, sid, 16)
    _fill_ones_rows(upd, _CH, 16)
    _fill_ones_rows(upd_r, _DEG_REM, 16)
    plsc.subcore_barrier()

    ebase = (c * _NS + sid) * _DEG_EPT

    def chunk(i, _):
        pltpu.sync_copy(dst_hbm.at[pl.ds(ebase + i * _CH, _CH)], dst_buf)
        pltpu.sync_copy(upd, deg_sh.at[dst_buf], add=True)
        return 0

    lax.fori_loop(0, _DEG_FULL, chunk, 0)

    b = ebase + _DEG_FULL * _CH
    pltpu.sync_copy(dst_hbm.at[pl.ds(b, _DEG_REM)], dst_r)
    pltpu.sync_copy(upd_r, deg_sh.at[dst_r], add=True)

    plsc.subcore_barrier()
    base = sid * _ROWS_PT
    pltpu.sync_copy(deg_sh.at[pl.ds(base, _ROWS_PT)],
                    out_hbm.at[c, pl.ds(base, _ROWS_PT)])


_deg_call = pl.kernel(
    _deg_body,
    out_type=jax.ShapeDtypeStruct((_NC, _N, 16), jnp.float32),
    mesh=_mesh,
    scratch_types=[
        pltpu.VMEM_SHARED((_N, 16), jnp.float32),
        pltpu.VMEM((_CH, 16), jnp.float32),
        pltpu.VMEM((_DEG_REM, 16), jnp.float32),
        pltpu.VMEM((_CH,), jnp.int32),
        pltpu.VMEM((_DEG_REM,), jnp.int32),
        pltpu.SemaphoreType.DMA,
    ],
)


_AGG_EPT = _E // _NS
_AGG_FULL = _AGG_EPT // _CH
_AGG_REM = _AGG_EPT - _AGG_FULL * _CH


def _agg_body(p2_hbm, src_hbm, dst_hbm, out_hbm,
              s_sh, rows, rows_r, src_buf, dst_buf, gidx,
              src_r, dst_r, gidx_r, sem):
    c = lax.axis_index("c")
    sid = lax.axis_index("s")

    _zero_vmem_rows(rows, _CH, _HALF)
    _zero_my_shared_slice(s_sh, rows, sid, _HALF)
    plsc.subcore_barrier()

    ebase = sid * _AGG_EPT

    def chunk(i, _):
        b = ebase + i * _CH
        pltpu.sync_copy(src_hbm.at[pl.ds(b, _CH)], src_buf)
        pltpu.sync_copy(dst_hbm.at[pl.ds(b, _CH)], dst_buf)

        def cvt(k, _):
            v = src_buf[pl.ds(k * 16, 16)]
            gidx[pl.ds(k * 16, 16)] = v * 2 + c
            return 0

        lax.fori_loop(0, _CH // 16, cvt, 0)
        pltpu.async_copy(p2_hbm.at[gidx], rows, sem).wait()
        pltpu.sync_copy(rows, s_sh.at[dst_buf], add=True)
        return 0

    lax.fori_loop(0, _AGG_FULL, chunk, 0)

    b = ebase + _AGG_FULL * _CH
    pltpu.sync_copy(src_hbm.at[pl.ds(b, _AGG_REM)], src_r)
    pltpu.sync_copy(dst_hbm.at[pl.ds(b, _AGG_REM)], dst_r)
    for k in range(_AGG_REM // 16):
        v = src_r[pl.ds(k * 16, 16)]
        gidx_r[pl.ds(k * 16, 16)] = v * 2 + c
    pltpu.async_copy(p2_hbm.at[gidx_r], rows_r, sem).wait()
    pltpu.sync_copy(rows_r, s_sh.at[dst_r], add=True)

    plsc.subcore_barrier()
    base = sid * _ROWS_PT
    pltpu.sync_copy(s_sh.at[pl.ds(base, _ROWS_PT)],
                    out_hbm.at[c, pl.ds(base, _ROWS_PT)])


_agg_call = pl.kernel(
    _agg_body,
    out_type=jax.ShapeDtypeStruct((_NC, _N, _HALF), jnp.float32),
    mesh=_mesh,
    scratch_types=[
        pltpu.VMEM_SHARED((_N, _HALF), jnp.float32),
        pltpu.VMEM((_CH, _HALF), jnp.float32),
        pltpu.VMEM((_AGG_REM, _HALF), jnp.float32),
        pltpu.VMEM((_CH,), jnp.int32),
        pltpu.VMEM((_CH,), jnp.int32),
        pltpu.VMEM((_CH,), jnp.int32),
        pltpu.VMEM((_AGG_REM,), jnp.int32),
        pltpu.VMEM((_AGG_REM,), jnp.int32),
        pltpu.VMEM((_AGG_REM,), jnp.int32),
        pltpu.SemaphoreType.DMA,
    ],
)


_BS = 2000
_NG = _N // _BS


def _k0_body(x_ref, w_ref, dinv_ref, o_ref):
    o_ref[...] = (
        jnp.dot(x_ref[...], w_ref[...], preferred_element_type=jnp.float32)
        * dinv_ref[...]
    )


def _mid_body(has_res, *refs):
    if has_res:
        s_ref, p_ref, dinv_ref, gs_ref, sh_ref, res_ref, w_ref, z_ref, pn_ref = refs
    else:
        s_ref, p_ref, dinv_ref, gs_ref, sh_ref, w_ref, z_ref, pn_ref = refs
    s_cat = jnp.concatenate([s_ref[0], s_ref[1]], axis=1)
    a = (s_cat + p_ref[...]) * dinv_ref[...]
    z = jnp.maximum(a * gs_ref[...] + sh_ref[...], 0.0)
    if has_res:
        z = z + res_ref[...]
    z_ref[...] = z
    pn_ref[...] = (
        jnp.dot(z, w_ref[...], preferred_element_type=jnp.float32) * dinv_ref[...]
    )


def _fin_body(s_ref, p_ref, dinv_ref, gs_ref, sh_ref, res_ref, wc_ref, bc_ref,
              acc_ref, o_ref):
    i = pl.program_id(0)
    s_cat = jnp.concatenate([s_ref[0], s_ref[1]], axis=1)
    a = (s_cat + p_ref[...]) * dinv_ref[...]
    z = jnp.maximum(a * gs_ref[...] + sh_ref[...], 0.0) + res_ref[...]
    part = jnp.sum(z, axis=0, keepdims=True)

    @pl.when(i == 0)
    def _():
        acc_ref[...] = part

    @pl.when(i > 0)
    def _():
        acc_ref[...] = acc_ref[...] + part

    @pl.when(i == pl.num_programs(0) - 1)
    def _():
        pooled = acc_ref[...] * (1.0 / _N)
        o_ref[...] = (
            jnp.dot(pooled, wc_ref[...], preferred_element_type=jnp.float32)
            + bc_ref[...]
        )


def _row_spec(w):
    return pl.BlockSpec((_BS, w), lambda i: (i, 0))


def _fixed_spec(r, w):
    return pl.BlockSpec((r, w), lambda i: (0, 0))


_S_SPEC = pl.BlockSpec((_NC, _BS, _HALF), lambda i: (0, i, 0))

_k0 = pl.pallas_call(
    _k0_body,
    grid=(_NG,),
    in_specs=[_row_spec(_FIN), _fixed_spec(_FIN, _H), _row_spec(1)],
    out_specs=_row_spec(_H),
    out_shape=jax.ShapeDtypeStruct((_N, _H), jnp.float32),
)

_k_mid1 = pl.pallas_call(
    functools.partial(_mid_body, False),
    grid=(_NG,),
    in_specs=[_S_SPEC, _row_spec(_H), _row_spec(1), _fixed_spec(1, _H),
              _fixed_spec(1, _H), _fixed_spec(_H, _H)],
    out_specs=[_row_spec(_H), _row_spec(_H)],
    out_shape=[jax.ShapeDtypeStruct((_N, _H), jnp.float32),
               jax.ShapeDtypeStruct((_N, _H), jnp.float32)],
)

_k_mid2 = pl.pallas_call(
    functools.partial(_mid_body, True),
    grid=(_NG,),
    in_specs=[_S_SPEC, _row_spec(_H), _row_spec(1), _fixed_spec(1, _H),
              _fixed_spec(1, _H), _row_spec(_H), _fixed_spec(_H, _H)],
    out_specs=[_row_spec(_H), _row_spec(_H)],
    out_shape=[jax.ShapeDtypeStruct((_N, _H), jnp.float32),
               jax.ShapeDtypeStruct((_N, _H), jnp.float32)],
)

_k_fin = pl.pallas_call(
    _fin_body,
    grid=(_NG,),
    in_specs=[_S_SPEC, _row_spec(_H), _row_spec(1), _fixed_spec(1, _H),
              _fixed_spec(1, _H), _row_spec(_H), _fixed_spec(_H, _C),
              _fixed_spec(1, _C)],
    out_specs=[_fixed_spec(1, _H), _fixed_spec(1, _C)],
    out_shape=[jax.ShapeDtypeStruct((1, _H), jnp.float32),
               jax.ShapeDtypeStruct((1, _C), jnp.float32)],
)


def kernel(x, edge_index, W0, b0, W1, b1, W2, b2, g0, be0, g1, be1, g2, be2,
           Wc, bc):
    src = edge_index[0]
    dst = edge_index[1]

    degp = _deg_call(dst)
    deg = degp[0, :, 0] + degp[1, :, 0] + 1.0
    dinv = jax.lax.rsqrt(deg)[:, None]

    kappa = 1.0 / jnp.sqrt(jnp.float32(1.0) + _EPS)
    gs0, gs1, gs2 = g0 * kappa, g1 * kappa, g2 * kappa
    sh0 = (b0 * gs0 + be0)[None, :]
    sh1 = (b1 * gs1 + be1)[None, :]
    sh2 = (b2 * gs2 + be2)[None, :]
    gs0, gs1, gs2 = gs0[None, :], gs1[None, :], gs2[None, :]

    p = _k0(x, W0, dinv)
    s = _agg_call(p.reshape(2 * _N, _HALF), src, dst)
    z1, p = _k_mid1(s, p, dinv, gs0, sh0, W1)
    s = _agg_call(p.reshape(2 * _N, _HALF), src, dst)
    z2, p = _k_mid2(s, p, dinv, gs1, sh1, z1, W2)
    s = _agg_call(p.reshape(2 * _N, _HALF), src, dst)
    _, out = _k_fin(s, p, dinv, gs2, sh2, z2, Wc, bc[None, :])
    return out


# speedup vs baseline: 9.7688x; 9.7688x over previous
import functools

import jax
import jax.numpy as jnp
from jax import lax
from jax.experimental import pallas as pl
from jax.experimental.pallas import tpu as pltpu
from jax.experimental.pallas import tpu_sc as plsc

_N = 10000
_E = 320000
_FIN = 128
_H = 256
_HALF = 128
_C = 10
_EPS = 1e-5

_NC = 2
_NS = 16
_CH = 128

_RPT = 624
_RXTRA = _N - _RPT * _NS

_mesh = plsc.VectorSubcoreMesh(
    core_axis_name="c", subcore_axis_name="s", num_cores=_NC, num_subcores=_NS
)


def _zero_vmem_rows(buf, nrows, width):
    def body(i, _):
        for j in range(width // 16):
            buf[i, pl.ds(j * 16, 16)] = jnp.zeros((16,), jnp.float32)
        return 0

    lax.fori_loop(0, nrows, body, 0)


def _fill_ones_rows(buf, nrows, width):
    def body(i, _):
        for j in range(width // 16):
            buf[i, pl.ds(j * 16, 16)] = jnp.ones((16,), jnp.float32)
        return 0

    lax.fori_loop(0, nrows, body, 0)


def _zero_my_shared_slice(acc_sh, zbuf, sid):
    base = sid * _RPT
    for off, size in ((0, 128), (128, 128), (256, 128), (384, 128), (512, 112)):
        pltpu.sync_copy(zbuf.at[pl.ds(0, size)], acc_sh.at[pl.ds(base + off, size)])

    @pl.when(sid == _NS - 1)
    def _():
        pltpu.sync_copy(zbuf.at[pl.ds(0, _RXTRA)],
                        acc_sh.at[pl.ds(_RPT * _NS, _RXTRA)])


def _writeout_my_shared_slice(acc_sh, out_hbm, c, sid):
    base = sid * _RPT
    pltpu.sync_copy(acc_sh.at[pl.ds(base, _RPT)],
                    out_hbm.at[c, pl.ds(base, _RPT)])

    @pl.when(sid == _NS - 1)
    def _():
        pltpu.sync_copy(acc_sh.at[pl.ds(_RPT * _NS, _RXTRA)],
                        out_hbm.at[c, pl.ds(_RPT * _NS, _RXTRA)])


_DEG_EPT = _E // (_NC * _NS)
_DEG_FULL = _DEG_EPT // _CH
_DEG_REM = _DEG_EPT - _DEG_FULL * _CH


def _deg_body(dst_hbm, out_hbm, deg_sh, upd, upd_r, dst_buf, dst_r, sem):
    c = lax.axis_index("c")
    sid = lax.axis_index("s")

    _zero_vmem_rows(upd, _CH, 16)
    _zero_my_shared_slice(deg_sh, upd, sid)
    _fill_ones_rows(upd, _CH, 16)
    _fill_ones_rows(upd_r, _DEG_REM, 16)
    plsc.subcore_barrier()

    ebase = (c * _NS + sid) * _DEG_EPT

    def chunk(i, _):
        pltpu.sync_copy(dst_hbm.at[pl.ds(ebase + i * _CH, _CH)], dst_buf)
        pltpu.sync_copy(upd, deg_sh.at[dst_buf], add=True)
        return 0

    lax.fori_loop(0, _DEG_FULL, chunk, 0)

    b = ebase + _DEG_FULL * _CH
    pltpu.sync_copy(dst_hbm.at[pl.ds(b, _DEG_REM)], dst_r)
    pltpu.sync_copy(upd_r, deg_sh.at[dst_r], add=True)

    plsc.subcore_barrier()
    _writeout_my_shared_slice(deg_sh, out_hbm, c, sid)


_deg_call = pl.kernel(
    _deg_body,
    out_type=jax.ShapeDtypeStruct((_NC, _N, 16), jnp.float32),
    mesh=_mesh,
    scratch_types=[
        pltpu.VMEM_SHARED((_N, 16), jnp.float32),
        pltpu.VMEM((_CH, 16), jnp.float32),
        pltpu.VMEM((_DEG_REM, 16), jnp.float32),
        pltpu.VMEM((_CH,), jnp.int32),
        pltpu.VMEM((_DEG_REM,), jnp.int32),
        pltpu.SemaphoreType.DMA,
    ],
)


_AGG_EPT = _E // _NS
_AGG_FULL = _AGG_EPT // _CH
_AGG_REM = _AGG_EPT - _AGG_FULL * _CH


def _agg_body(p2_hbm, src_hbm, dst_hbm, out_hbm,
              s_sh, rows, rows_r, src_buf, dst_buf, gidx,
              src_r, dst_r, gidx_r, sem):
    c = lax.axis_index("c")
    sid = lax.axis_index("s")

    _zero_vmem_rows(rows, _CH, _HALF)
    _zero_my_shared_slice(s_sh, rows, sid)
    plsc.subcore_barrier()

    ebase = sid * _AGG_EPT

    def chunk(i, _):
        b = ebase + i * _CH
        pltpu.sync_copy(src_hbm.at[pl.ds(b, _CH)], src_buf)
        pltpu.sync_copy(dst_hbm.at[pl.ds(b, _CH)], dst_buf)

        def cvt(k, _):
            v = src_buf[pl.ds(k * 16, 16)]
            gidx[pl.ds(k * 16, 16)] = v * 2 + c
            return 0

        lax.fori_loop(0, _CH // 16, cvt, 0)
        pltpu.async_copy(p2_hbm.at[gidx], rows, sem).wait()
        pltpu.sync_copy(rows, s_sh.at[dst_buf], add=True)
        return 0

    lax.fori_loop(0, _AGG_FULL, chunk, 0)

    b = ebase + _AGG_FULL * _CH
    pltpu.sync_copy(src_hbm.at[pl.ds(b, _AGG_REM)], src_r)
    pltpu.sync_copy(dst_hbm.at[pl.ds(b, _AGG_REM)], dst_r)
    for k in range(_AGG_REM // 16):
        v = src_r[pl.ds(k * 16, 16)]
        gidx_r[pl.ds(k * 16, 16)] = v * 2 + c
    pltpu.async_copy(p2_hbm.at[gidx_r], rows_r, sem).wait()
    pltpu.sync_copy(rows_r, s_sh.at[dst_r], add=True)

    plsc.subcore_barrier()
    _writeout_my_shared_slice(s_sh, out_hbm, c, sid)


_agg_call = pl.kernel(
    _agg_body,
    out_type=jax.ShapeDtypeStruct((_NC, _N, _HALF), jnp.float32),
    mesh=_mesh,
    scratch_types=[
        pltpu.VMEM_SHARED((_N, _HALF), jnp.float32),
        pltpu.VMEM((_CH, _HALF), jnp.float32),
        pltpu.VMEM((_AGG_REM, _HALF), jnp.float32),
        pltpu.VMEM((_CH,), jnp.int32),
        pltpu.VMEM((_CH,), jnp.int32),
        pltpu.VMEM((_CH,), jnp.int32),
        pltpu.VMEM((_AGG_REM,), jnp.int32),
        pltpu.VMEM((_AGG_REM,), jnp.int32),
        pltpu.VMEM((_AGG_REM,), jnp.int32),
        pltpu.SemaphoreType.DMA,
    ],
)


_BS = 2000
_NG = _N // _BS


def _k0_body(x_ref, w_ref, dinv_ref, o_ref):
    o_ref[...] = (
        jnp.dot(x_ref[...], w_ref[...], preferred_element_type=jnp.float32)
        * dinv_ref[...]
    )


def _mid_body(has_res, *refs):
    if has_res:
        s_ref, p_ref, dinv_ref, gs_ref, sh_ref, res_ref, w_ref, z_ref, pn_ref = refs
    else:
        s_ref, p_ref, dinv_ref, gs_ref, sh_ref, w_ref, z_ref, pn_ref = refs
    s_cat = jnp.concatenate([s_ref[0], s_ref[1]], axis=1)
    a = (s_cat + p_ref[...]) * dinv_ref[...]
    z = jnp.maximum(a * gs_ref[...] + sh_ref[...], 0.0)
    if has_res:
        z = z + res_ref[...]
    z_ref[...] = z
    pn_ref[...] = (
        jnp.dot(z, w_ref[...], preferred_element_type=jnp.float32) * dinv_ref[...]
    )


def _fin_body(s_ref, p_ref, dinv_ref, gs_ref, sh_ref, res_ref, wc_ref, bc_ref,
              acc_ref, o_ref):
    i = pl.program_id(0)
    s_cat = jnp.concatenate([s_ref[0], s_ref[1]], axis=1)
    a = (s_cat + p_ref[...]) * dinv_ref[...]
    z = jnp.maximum(a * gs_ref[...] + sh_ref[...], 0.0) + res_ref[...]
    part = jnp.sum(z, axis=0, keepdims=True)

    @pl.when(i == 0)
    def _():
        acc_ref[...] = part

    @pl.when(i > 0)
    def _():
        acc_ref[...] = acc_ref[...] + part

    @pl.when(i == pl.num_programs(0) - 1)
    def _():
        pooled = acc_ref[...] * (1.0 / _N)
        o_ref[...] = (
            jnp.dot(pooled, wc_ref[...], preferred_element_type=jnp.float32)
            + bc_ref[...]
        )


def _row_spec(w):
    return pl.BlockSpec((_BS, w), lambda i: (i, 0))


def _fixed_spec(r, w):
    return pl.BlockSpec((r, w), lambda i: (0, 0))


_S_SPEC = pl.BlockSpec((_NC, _BS, _HALF), lambda i: (0, i, 0))

_k0 = pl.pallas_call(
    _k0_body,
    grid=(_NG,),
    in_specs=[_row_spec(_FIN), _fixed_spec(_FIN, _H), _row_spec(1)],
    out_specs=_row_spec(_H),
    out_shape=jax.ShapeDtypeStruct((_N, _H), jnp.float32),
)

_k_mid1 = pl.pallas_call(
    functools.partial(_mid_body, False),
    grid=(_NG,),
    in_specs=[_S_SPEC, _row_spec(_H), _row_spec(1), _fixed_spec(1, _H),
              _fixed_spec(1, _H), _fixed_spec(_H, _H)],
    out_specs=[_row_spec(_H), _row_spec(_H)],
    out_shape=[jax.ShapeDtypeStruct((_N, _H), jnp.float32),
               jax.ShapeDtypeStruct((_N, _H), jnp.float32)],
)

_k_mid2 = pl.pallas_call(
    functools.partial(_mid_body, True),
    grid=(_NG,),
    in_specs=[_S_SPEC, _row_spec(_H), _row_spec(1), _fixed_spec(1, _H),
              _fixed_spec(1, _H), _row_spec(_H), _fixed_spec(_H, _H)],
    out_specs=[_row_spec(_H), _row_spec(_H)],
    out_shape=[jax.ShapeDtypeStruct((_N, _H), jnp.float32),
               jax.ShapeDtypeStruct((_N, _H), jnp.float32)],
)

_k_fin = pl.pallas_call(
    _fin_body,
    grid=(_NG,),
    in_specs=[_S_SPEC, _row_spec(_H), _row_spec(1), _fixed_spec(1, _H),
              _fixed_spec(1, _H), _row_spec(_H), _fixed_spec(_H, _C),
              _fixed_spec(1, _C)],
    out_specs=[_fixed_spec(1, _H), _fixed_spec(1, _C)],
    out_shape=[jax.ShapeDtypeStruct((1, _H), jnp.float32),
               jax.ShapeDtypeStruct((1, _C), jnp.float32)],
)


def kernel(x, edge_index, W0, b0, W1, b1, W2, b2, g0, be0, g1, be1, g2, be2,
           Wc, bc):
    src = edge_index[0]
    dst = edge_index[1]

    degp = _deg_call(dst)
    deg = degp[0, :, 0] + degp[1, :, 0] + 1.0
    dinv = jax.lax.rsqrt(deg)[:, None]

    kappa = 1.0 / jnp.sqrt(jnp.float32(1.0) + _EPS)
    gs0, gs1, gs2 = g0 * kappa, g1 * kappa, g2 * kappa
    sh0 = (b0 * gs0 + be0)[None, :]
    sh1 = (b1 * gs1 + be1)[None, :]
    sh2 = (b2 * gs2 + be2)[None, :]
    gs0, gs1, gs2 = gs0[None, :], gs1[None, :], gs2[None, :]

    p = _k0(x, W0, dinv)
    s = _agg_call(p.reshape(2 * _N, _HALF), src, dst)
    z1, p = _k_mid1(s, p, dinv, gs0, sh0, W1)
    s = _agg_call(p.reshape(2 * _N, _HALF), src, dst)
    z2, p = _k_mid2(s, p, dinv, gs1, sh1, z1, W2)
    s = _agg_call(p.reshape(2 * _N, _HALF), src, dst)
    _, out = _k_fin(s, p, dinv, gs2, sh2, z2, Wc, bc[None, :])
    return out


# speedup vs baseline: 15.1139x; 1.5472x over previous
import functools

import jax
import jax.numpy as jnp
from jax import lax
from jax.experimental import pallas as pl
from jax.experimental.pallas import tpu as pltpu
from jax.experimental.pallas import tpu_sc as plsc

_N = 10000
_E = 320000
_FIN = 128
_H = 256
_HALF = 128
_C = 10
_EPS = 1e-5

_NC = 2
_NS = 16
_CH = 128

_RPT = 624
_RXTRA = _N - _RPT * _NS

_mesh = plsc.VectorSubcoreMesh(
    core_axis_name="c", subcore_axis_name="s", num_cores=_NC, num_subcores=_NS
)


def _zero_vmem_rows(buf, nrows, width):
    def body(i, _):
        for j in range(width // 16):
            buf[i, pl.ds(j * 16, 16)] = jnp.zeros((16,), jnp.float32)
        return 0

    lax.fori_loop(0, nrows, body, 0)


def _fill_ones_rows(buf, nrows, width):
    def body(i, _):
        for j in range(width // 16):
            buf[i, pl.ds(j * 16, 16)] = jnp.ones((16,), jnp.float32)
        return 0

    lax.fori_loop(0, nrows, body, 0)


def _zero_my_shared_slice(acc_sh, zbuf, sid):
    base = sid * _RPT
    for off, size in ((0, 128), (128, 128), (256, 128), (384, 128), (512, 112)):
        pltpu.sync_copy(zbuf.at[pl.ds(0, size)], acc_sh.at[pl.ds(base + off, size)])

    @pl.when(sid == _NS - 1)
    def _():
        pltpu.sync_copy(zbuf.at[pl.ds(0, _RXTRA)],
                        acc_sh.at[pl.ds(_RPT * _NS, _RXTRA)])


def _writeout_my_shared_slice(acc_sh, out_hbm, c, sid):
    base = sid * _RPT
    pltpu.sync_copy(acc_sh.at[pl.ds(base, _RPT)],
                    out_hbm.at[c, pl.ds(base, _RPT)])

    @pl.when(sid == _NS - 1)
    def _():
        pltpu.sync_copy(acc_sh.at[pl.ds(_RPT * _NS, _RXTRA)],
                        out_hbm.at[c, pl.ds(_RPT * _NS, _RXTRA)])


_DEG_EPT = _E // (_NC * _NS)
_DEG_FULL = _DEG_EPT // _CH
_DEG_REM = _DEG_EPT - _DEG_FULL * _CH


def _deg_body(dst_hbm, out_hbm, deg_sh, upd, upd_r, dst_buf, dst_r, sem):
    c = lax.axis_index("c")
    sid = lax.axis_index("s")

    _zero_vmem_rows(upd, _CH, 16)
    _zero_my_shared_slice(deg_sh, upd, sid)
    _fill_ones_rows(upd, _CH, 16)
    _fill_ones_rows(upd_r, _DEG_REM, 16)
    plsc.subcore_barrier()

    ebase = (c * _NS + sid) * _DEG_EPT

    def chunk(i, _):
        pltpu.sync_copy(dst_hbm.at[pl.ds(ebase + i * _CH, _CH)], dst_buf)
        pltpu.sync_copy(upd, deg_sh.at[dst_buf], add=True)
        return 0

    lax.fori_loop(0, _DEG_FULL, chunk, 0)

    b = ebase + _DEG_FULL * _CH
    pltpu.sync_copy(dst_hbm.at[pl.ds(b, _DEG_REM)], dst_r)
    pltpu.sync_copy(upd_r, deg_sh.at[dst_r], add=True)

    plsc.subcore_barrier()
    _writeout_my_shared_slice(deg_sh, out_hbm, c, sid)


_deg_call = pl.kernel(
    _deg_body,
    out_type=jax.ShapeDtypeStruct((_NC, _N, 16), jnp.float32),
    mesh=_mesh,
    scratch_types=[
        pltpu.VMEM_SHARED((_N, 16), jnp.float32),
        pltpu.VMEM((_CH, 16), jnp.float32),
        pltpu.VMEM((_DEG_REM, 16), jnp.float32),
        pltpu.VMEM((_CH,), jnp.int32),
        pltpu.VMEM((_DEG_REM,), jnp.int32),
        pltpu.SemaphoreType.DMA,
    ],
)


_AGG_EPT = _E // _NS
_AGG_FULL = _AGG_EPT // _CH
_AGG_REM = _AGG_EPT - _AGG_FULL * _CH


def _agg_body(p2_hbm, src_hbm, dst_hbm, out_hbm,
              s_sh, rows_a, rows_b, rows_r,
              src_a, dst_a, gidx_a, src_b, dst_b, gidx_b,
              src_r, dst_r, gidx_r, sem_a, sem_b):
    c = lax.axis_index("c")
    sid = lax.axis_index("s")

    _zero_vmem_rows(rows_a, _CH, _HALF)
    _zero_my_shared_slice(s_sh, rows_a, sid)
    plsc.subcore_barrier()

    ebase = sid * _AGG_EPT

    def load_and_start(ci, sbuf, dbuf, gbuf, rbuf, sem):
        b = ebase + ci * _CH
        pltpu.sync_copy(src_hbm.at[pl.ds(b, _CH)], sbuf)
        pltpu.sync_copy(dst_hbm.at[pl.ds(b, _CH)], dbuf)

        def cvt(k, _):
            v = sbuf[pl.ds(k * 16, 16)]
            gbuf[pl.ds(k * 16, 16)] = v * 2 + c
            return 0

        lax.fori_loop(0, _CH // 16, cvt, 0)
        pltpu.async_copy(p2_hbm.at[gbuf], rbuf, sem)

    load_and_start(0, src_a, dst_a, gidx_a, rows_a, sem_a)

    def pair(k, _):
        load_and_start(2 * k + 1, src_b, dst_b, gidx_b, rows_b, sem_b)
        pltpu.make_async_copy(p2_hbm.at[gidx_a], rows_a, sem_a).wait()
        pltpu.sync_copy(rows_a, s_sh.at[dst_a], add=True)

        @pl.when(k < _AGG_FULL // 2 - 1)
        def _():
            load_and_start(2 * k + 2, src_a, dst_a, gidx_a, rows_a, sem_a)

        pltpu.make_async_copy(p2_hbm.at[gidx_b], rows_b, sem_b).wait()
        pltpu.sync_copy(rows_b, s_sh.at[dst_b], add=True)
        return 0

    lax.fori_loop(0, _AGG_FULL // 2, pair, 0)

    b = ebase + _AGG_FULL * _CH
    pltpu.sync_copy(src_hbm.at[pl.ds(b, _AGG_REM)], src_r)
    pltpu.sync_copy(dst_hbm.at[pl.ds(b, _AGG_REM)], dst_r)
    for k in range(_AGG_REM // 16):
        v = src_r[pl.ds(k * 16, 16)]
        gidx_r[pl.ds(k * 16, 16)] = v * 2 + c
    pltpu.async_copy(p2_hbm.at[gidx_r], rows_r, sem_a).wait()
    pltpu.sync_copy(rows_r, s_sh.at[dst_r], add=True)

    plsc.subcore_barrier()
    _writeout_my_shared_slice(s_sh, out_hbm, c, sid)


_agg_call = pl.kernel(
    _agg_body,
    out_type=jax.ShapeDtypeStruct((_NC, _N, _HALF), jnp.float32),
    mesh=_mesh,
    scratch_types=[
        pltpu.VMEM_SHARED((_N, _HALF), jnp.float32),
        pltpu.VMEM((_CH, _HALF), jnp.float32),
        pltpu.VMEM((_CH, _HALF), jnp.float32),
        pltpu.VMEM((_AGG_REM, _HALF), jnp.float32),
        pltpu.VMEM((_CH,), jnp.int32),
        pltpu.VMEM((_CH,), jnp.int32),
        pltpu.VMEM((_CH,), jnp.int32),
        pltpu.VMEM((_CH,), jnp.int32),
        pltpu.VMEM((_CH,), jnp.int32),
        pltpu.VMEM((_CH,), jnp.int32),
        pltpu.VMEM((_AGG_REM,), jnp.int32),
        pltpu.VMEM((_AGG_REM,), jnp.int32),
        pltpu.VMEM((_AGG_REM,), jnp.int32),
        pltpu.SemaphoreType.DMA,
        pltpu.SemaphoreType.DMA,
    ],
)


_BS = 2000
_NG = _N // _BS


def _k0_body(x_ref, w_ref, dinv_ref, o_ref):
    o_ref[...] = (
        jnp.dot(x_ref[...], w_ref[...], preferred_element_type=jnp.float32)
        * dinv_ref[...]
    )


def _mid_body(has_res, *refs):
    if has_res:
        s_ref, p_ref, dinv_ref, gs_ref, sh_ref, res_ref, w_ref, z_ref, pn_ref = refs
    else:
        s_ref, p_ref, dinv_ref, gs_ref, sh_ref, w_ref, z_ref, pn_ref = refs
    s_cat = jnp.concatenate([s_ref[0], s_ref[1]], axis=1)
    a = (s_cat + p_ref[...]) * dinv_ref[...]
    z = jnp.maximum(a * gs_ref[...] + sh_ref[...], 0.0)
    if has_res:
        z = z + res_ref[...]
    z_ref[...] = z
    pn_ref[...] = (
        jnp.dot(z, w_ref[...], preferred_element_type=jnp.float32) * dinv_ref[...]
    )


def _fin_body(s_ref, p_ref, dinv_ref, gs_ref, sh_ref, res_ref, wc_ref, bc_ref,
              acc_ref, o_ref):
    i = pl.program_id(0)
    s_cat = jnp.concatenate([s_ref[0], s_ref[1]], axis=1)
    a = (s_cat + p_ref[...]) * dinv_ref[...]
    z = jnp.maximum(a * gs_ref[...] + sh_ref[...], 0.0) + res_ref[...]
    part = jnp.sum(z, axis=0, keepdims=True)

    @pl.when(i == 0)
    def _():
        acc_ref[...] = part

    @pl.when(i > 0)
    def _():
        acc_ref[...] = acc_ref[...] + part

    @pl.when(i == pl.num_programs(0) - 1)
    def _():
        pooled = acc_ref[...] * (1.0 / _N)
        o_ref[...] = (
            jnp.dot(pooled, wc_ref[...], preferred_element_type=jnp.float32)
            + bc_ref[...]
        )


def _row_spec(w):
    return pl.BlockSpec((_BS, w), lambda i: (i, 0))


def _fixed_spec(r, w):
    return pl.BlockSpec((r, w), lambda i: (0, 0))


_S_SPEC = pl.BlockSpec((_NC, _BS, _HALF), lambda i: (0, i, 0))

_k0 = pl.pallas_call(
    _k0_body,
    grid=(_NG,),
    in_specs=[_row_spec(_FIN), _fixed_spec(_FIN, _H), _row_spec(1)],
    out_specs=_row_spec(_H),
    out_shape=jax.ShapeDtypeStruct((_N, _H), jnp.float32),
)

_k_mid1 = pl.pallas_call(
    functools.partial(_mid_body, False),
    grid=(_NG,),
    in_specs=[_S_SPEC, _row_spec(_H), _row_spec(1), _fixed_spec(1, _H),
              _fixed_spec(1, _H), _fixed_spec(_H, _H)],
    out_specs=[_row_spec(_H), _row_spec(_H)],
    out_shape=[jax.ShapeDtypeStruct((_N, _H), jnp.float32),
               jax.ShapeDtypeStruct((_N, _H), jnp.float32)],
)

_k_mid2 = pl.pallas_call(
    functools.partial(_mid_body, True),
    grid=(_NG,),
    in_specs=[_S_SPEC, _row_spec(_H), _row_spec(1), _fixed_spec(1, _H),
              _fixed_spec(1, _H), _row_spec(_H), _fixed_spec(_H, _H)],
    out_specs=[_row_spec(_H), _row_spec(_H)],
    out_shape=[jax.ShapeDtypeStruct((_N, _H), jnp.float32),
               jax.ShapeDtypeStruct((_N, _H), jnp.float32)],
)

_k_fin = pl.pallas_call(
    _fin_body,
    grid=(_NG,),
    in_specs=[_S_SPEC, _row_spec(_H), _row_spec(1), _fixed_spec(1, _H),
              _fixed_spec(1, _H), _row_spec(_H), _fixed_spec(_H, _C),
              _fixed_spec(1, _C)],
    out_specs=[_fixed_spec(1, _H), _fixed_spec(1, _C)],
    out_shape=[jax.ShapeDtypeStruct((1, _H), jnp.float32),
               jax.ShapeDtypeStruct((1, _C), jnp.float32)],
)


def kernel(x, edge_index, W0, b0, W1, b1, W2, b2, g0, be0, g1, be1, g2, be2,
           Wc, bc):
    src = edge_index[0]
    dst = edge_index[1]

    degp = _deg_call(dst)
    deg = degp[0, :, 0] + degp[1, :, 0] + 1.0
    dinv = jax.lax.rsqrt(deg)[:, None]

    kappa = 1.0 / jnp.sqrt(jnp.float32(1.0) + _EPS)
    gs0, gs1, gs2 = g0 * kappa, g1 * kappa, g2 * kappa
    sh0 = (b0 * gs0 + be0)[None, :]
    sh1 = (b1 * gs1 + be1)[None, :]
    sh2 = (b2 * gs2 + be2)[None, :]
    gs0, gs1, gs2 = gs0[None, :], gs1[None, :], gs2[None, :]

    p = _k0(x, W0, dinv)
    s = _agg_call(p.reshape(2 * _N, _HALF), src, dst)
    z1, p = _k_mid1(s, p, dinv, gs0, sh0, W1)
    s = _agg_call(p.reshape(2 * _N, _HALF), src, dst)
    z2, p = _k_mid2(s, p, dinv, gs1, sh1, z1, W2)
    s = _agg_call(p.reshape(2 * _N, _HALF), src, dst)
    _, out = _k_fin(s, p, dinv, gs2, sh2, z2, Wc, bc[None, :])
    return out


# speedup vs baseline: 18.2214x; 1.2056x over previous
import functools

import jax
import jax.numpy as jnp
from jax import lax
from jax.experimental import pallas as pl
from jax.experimental.pallas import tpu as pltpu
from jax.experimental.pallas import tpu_sc as plsc

_N = 10000
_E = 320000
_FIN = 128
_H = 256
_HALF = 128
_C = 10
_EPS = 1e-5

_NC = 2
_NS = 16
_CH = 128

_RPT = 624
_RXTRA = _N - _RPT * _NS

_mesh = plsc.VectorSubcoreMesh(
    core_axis_name="c", subcore_axis_name="s", num_cores=_NC, num_subcores=_NS
)


def _zero_vmem_rows(buf, nrows, width):
    def body(i, _):
        for j in range(width // 16):
            buf[i, pl.ds(j * 16, 16)] = jnp.zeros((16,), jnp.float32)
        return 0

    lax.fori_loop(0, nrows, body, 0)


def _fill_ones_rows(buf, nrows, width):
    def body(i, _):
        for j in range(width // 16):
            buf[i, pl.ds(j * 16, 16)] = jnp.ones((16,), jnp.float32)
        return 0

    lax.fori_loop(0, nrows, body, 0)


def _zero_my_shared_slice(acc_sh, zbuf, sid):
    base = sid * _RPT
    for off, size in ((0, 128), (128, 128), (256, 128), (384, 128), (512, 112)):
        pltpu.sync_copy(zbuf.at[pl.ds(0, size)], acc_sh.at[pl.ds(base + off, size)])

    @pl.when(sid == _NS - 1)
    def _():
        pltpu.sync_copy(zbuf.at[pl.ds(0, _RXTRA)],
                        acc_sh.at[pl.ds(_RPT * _NS, _RXTRA)])


def _writeout_my_shared_slice(acc_sh, out_hbm, c, sid):
    base = sid * _RPT
    pltpu.sync_copy(acc_sh.at[pl.ds(base, _RPT)],
                    out_hbm.at[c, pl.ds(base, _RPT)])

    @pl.when(sid == _NS - 1)
    def _():
        pltpu.sync_copy(acc_sh.at[pl.ds(_RPT * _NS, _RXTRA)],
                        out_hbm.at[c, pl.ds(_RPT * _NS, _RXTRA)])


_DEG_EPT = _E // (_NC * _NS)
_DEG_FULL = _DEG_EPT // _CH
_DEG_REM = _DEG_EPT - _DEG_FULL * _CH


def _deg_body(dst_hbm, out_hbm, deg_sh, upd, upd_r, dst_buf, dst_r, sem):
    c = lax.axis_index("c")
    sid = lax.axis_index("s")

    _zero_vmem_rows(upd, _CH, 16)
    _zero_my_shared_slice(deg_sh, upd, sid)
    _fill_ones_rows(upd, _CH, 16)
    _fill_ones_rows(upd_r, _DEG_REM, 16)
    plsc.subcore_barrier()

    ebase = (c * _NS + sid) * _DEG_EPT

    def chunk(i, _):
        pltpu.sync_copy(dst_hbm.at[pl.ds(ebase + i * _CH, _CH)], dst_buf)
        pltpu.sync_copy(upd, deg_sh.at[dst_buf], add=True)
        return 0

    lax.fori_loop(0, _DEG_FULL, chunk, 0)

    b = ebase + _DEG_FULL * _CH
    pltpu.sync_copy(dst_hbm.at[pl.ds(b, _DEG_REM)], dst_r)
    pltpu.sync_copy(upd_r, deg_sh.at[dst_r], add=True)

    plsc.subcore_barrier()
    _writeout_my_shared_slice(deg_sh, out_hbm, c, sid)


_deg_call = pl.kernel(
    _deg_body,
    out_type=jax.ShapeDtypeStruct((_NC, _N, 16), jnp.float32),
    mesh=_mesh,
    scratch_types=[
        pltpu.VMEM_SHARED((_N, 16), jnp.float32),
        pltpu.VMEM((_CH, 16), jnp.float32),
        pltpu.VMEM((_DEG_REM, 16), jnp.float32),
        pltpu.VMEM((_CH,), jnp.int32),
        pltpu.VMEM((_DEG_REM,), jnp.int32),
        pltpu.SemaphoreType.DMA,
    ],
)


_NCHUNK = _E // _CH
_CPT = _NCHUNK // _NS
_XTRA = _NCHUNK - _CPT * _NS


def _agg_body(p2_hbm, src_hbm, dst_hbm, out_hbm, s_sh, *rest):
    slots = [rest[6 * i: 6 * i + 6] for i in range(3)]
    c = lax.axis_index("c")
    sid = lax.axis_index("s")

    rows0 = slots[0][3]

    def zrow(i, _):
        for j in range(_HALF // 16):
            rows0[i, pl.ds(j * 16, 16)] = jnp.zeros((16,), jnp.float32)
        return 0

    lax.fori_loop(0, _CH, zrow, 0)
    _zero_my_shared_slice(s_sh, rows0, sid)
    plsc.subcore_barrier()

    ebase = sid * _CPT * _CH

    def start(s_idx, slot):
        srcb, dstb, gi, rows, gsem, _ = slot
        b = ebase + s_idx * _CH
        pltpu.sync_copy(src_hbm.at[pl.ds(b, _CH)], srcb)
        pltpu.sync_copy(dst_hbm.at[pl.ds(b, _CH)], dstb)
        for k in range(_CH // 16):
            sl = pl.ds(k * 16, 16)
            gi[sl] = srcb[sl] * 2 + c
        pltpu.async_copy(p2_hbm.at[gi], rows, gsem)

    def drain(slot):
        _, dstb, gi, rows, gsem, ssem = slot
        pltpu.make_async_copy(p2_hbm.at[gi], rows, gsem).wait()
        pltpu.async_copy(rows, s_sh.at[dstb], ssem, add=True)

    def finish(slot):
        _, dstb, _, rows, _, ssem = slot
        pltpu.make_async_copy(rows, s_sh.at[dstb], ssem).wait()

    start(0, slots[0])
    start(1, slots[1])
    drain(slots[0])
    start(2, slots[2])
    drain(slots[1])

    def ring(k, _):
        for pi in range(3):
            s_idx = 3 * k + 3 + pi
            finish(slots[pi])
            start(s_idx, slots[pi])
            drain(slots[(pi + 2) % 3])
        return 0

    lax.fori_loop(0, _CPT // 3 - 1, ring, 0)
    drain(slots[2])
    for pi in range(3):
        finish(slots[pi])

    @pl.when(sid < _XTRA)
    def _():
        srcb, dstb, gi, rows, gsem, _ = slots[0]
        b = (_CPT * _NS + sid) * _CH
        pltpu.sync_copy(src_hbm.at[pl.ds(b, _CH)], srcb)
        pltpu.sync_copy(dst_hbm.at[pl.ds(b, _CH)], dstb)
        for k in range(_CH // 16):
            sl = pl.ds(k * 16, 16)
            gi[sl] = srcb[sl] * 2 + c
        pltpu.async_copy(p2_hbm.at[gi], rows, gsem).wait()
        pltpu.sync_copy(rows, s_sh.at[dstb], add=True)

    plsc.subcore_barrier()
    _writeout_my_shared_slice(s_sh, out_hbm, c, sid)


_agg_call = pl.kernel(
    _agg_body,
    out_type=jax.ShapeDtypeStruct((_NC, _N, _HALF), jnp.float32),
    mesh=_mesh,
    scratch_types=(
        [pltpu.VMEM_SHARED((_N, _HALF), jnp.float32)]
        + 3 * [
            pltpu.VMEM((_CH,), jnp.int32),
            pltpu.VMEM((_CH,), jnp.int32),
            pltpu.VMEM((_CH,), jnp.int32),
            pltpu.VMEM((_CH, _HALF), jnp.float32),
            pltpu.SemaphoreType.DMA,
            pltpu.SemaphoreType.DMA,
        ]
    ),
)


_BS = 2000
_NG = _N // _BS


def _k0_body(x_ref, w_ref, dinv_ref, o_ref):
    o_ref[...] = (
        jnp.dot(x_ref[...], w_ref[...], preferred_element_type=jnp.float32)
        * dinv_ref[...]
    )


def _mid_body(has_res, *refs):
    if has_res:
        s_ref, p_ref, dinv_ref, gs_ref, sh_ref, res_ref, w_ref, z_ref, pn_ref = refs
    else:
        s_ref, p_ref, dinv_ref, gs_ref, sh_ref, w_ref, z_ref, pn_ref = refs
    s_cat = jnp.concatenate([s_ref[0], s_ref[1]], axis=1)
    a = (s_cat + p_ref[...]) * dinv_ref[...]
    z = jnp.maximum(a * gs_ref[...] + sh_ref[...], 0.0)
    if has_res:
        z = z + res_ref[...]
    z_ref[...] = z
    pn_ref[...] = (
        jnp.dot(z, w_ref[...], preferred_element_type=jnp.float32) * dinv_ref[...]
    )


def _fin_body(s_ref, p_ref, dinv_ref, gs_ref, sh_ref, res_ref, wc_ref, bc_ref,
              acc_ref, o_ref):
    i = pl.program_id(0)
    s_cat = jnp.concatenate([s_ref[0], s_ref[1]], axis=1)
    a = (s_cat + p_ref[...]) * dinv_ref[...]
    z = jnp.maximum(a * gs_ref[...] + sh_ref[...], 0.0) + res_ref[...]
    part = jnp.sum(z, axis=0, keepdims=True)

    @pl.when(i == 0)
    def _():
        acc_ref[...] = part

    @pl.when(i > 0)
    def _():
        acc_ref[...] = acc_ref[...] + part

    @pl.when(i == pl.num_programs(0) - 1)
    def _():
        pooled = acc_ref[...] * (1.0 / _N)
        o_ref[...] = (
            jnp.dot(pooled, wc_ref[...], preferred_element_type=jnp.float32)
            + bc_ref[...]
        )


def _row_spec(w):
    return pl.BlockSpec((_BS, w), lambda i: (i, 0))


def _fixed_spec(r, w):
    return pl.BlockSpec((r, w), lambda i: (0, 0))


_S_SPEC = pl.BlockSpec((_NC, _BS, _HALF), lambda i: (0, i, 0))

_k0 = pl.pallas_call(
    _k0_body,
    grid=(_NG,),
    in_specs=[_row_spec(_FIN), _fixed_spec(_FIN, _H), _row_spec(1)],
    out_specs=_row_spec(_H),
    out_shape=jax.ShapeDtypeStruct((_N, _H), jnp.float32),
)

_k_mid1 = pl.pallas_call(
    functools.partial(_mid_body, False),
    grid=(_NG,),
    in_specs=[_S_SPEC, _row_spec(_H), _row_spec(1), _fixed_spec(1, _H),
              _fixed_spec(1, _H), _fixed_spec(_H, _H)],
    out_specs=[_row_spec(_H), _row_spec(_H)],
    out_shape=[jax.ShapeDtypeStruct((_N, _H), jnp.float32),
               jax.ShapeDtypeStruct((_N, _H), jnp.float32)],
)

_k_mid2 = pl.pallas_call(
    functools.partial(_mid_body, True),
    grid=(_NG,),
    in_specs=[_S_SPEC, _row_spec(_H), _row_spec(1), _fixed_spec(1, _H),
              _fixed_spec(1, _H), _row_spec(_H), _fixed_spec(_H, _H)],
    out_specs=[_row_spec(_H), _row_spec(_H)],
    out_shape=[jax.ShapeDtypeStruct((_N, _H), jnp.float32),
               jax.ShapeDtypeStruct((_N, _H), jnp.float32)],
)

_k_fin = pl.pallas_call(
    _fin_body,
    grid=(_NG,),
    in_specs=[_S_SPEC, _row_spec(_H), _row_spec(1), _fixed_spec(1, _H),
              _fixed_spec(1, _H), _row_spec(_H), _fixed_spec(_H, _C),
              _fixed_spec(1, _C)],
    out_specs=[_fixed_spec(1, _H), _fixed_spec(1, _C)],
    out_shape=[jax.ShapeDtypeStruct((1, _H), jnp.float32),
               jax.ShapeDtypeStruct((1, _C), jnp.float32)],
)


def kernel(x, edge_index, W0, b0, W1, b1, W2, b2, g0, be0, g1, be1, g2, be2,
           Wc, bc):
    src = edge_index[0]
    dst = edge_index[1]

    degp = _deg_call(dst)
    deg = degp[0, :, 0] + degp[1, :, 0] + 1.0
    dinv = jax.lax.rsqrt(deg)[:, None]

    kappa = 1.0 / jnp.sqrt(jnp.float32(1.0) + _EPS)
    gs0, gs1, gs2 = g0 * kappa, g1 * kappa, g2 * kappa
    sh0 = (b0 * gs0 + be0)[None, :]
    sh1 = (b1 * gs1 + be1)[None, :]
    sh2 = (b2 * gs2 + be2)[None, :]
    gs0, gs1, gs2 = gs0[None, :], gs1[None, :], gs2[None, :]

    p = _k0(x, W0, dinv)
    s = _agg_call(p.reshape(2 * _N, _HALF), src, dst)
    z1, p = _k_mid1(s, p, dinv, gs0, sh0, W1)
    s = _agg_call(p.reshape(2 * _N, _HALF), src, dst)
    z2, p = _k_mid2(s, p, dinv, gs1, sh1, z1, W2)
    s = _agg_call(p.reshape(2 * _N, _HALF), src, dst)
    _, out = _k_fin(s, p, dinv, gs2, sh2, z2, Wc, bc[None, :])
    return out


# speedup vs baseline: 19.6712x; 1.0796x over previous
import functools

import jax
import jax.numpy as jnp
from jax import lax
from jax.experimental import pallas as pl
from jax.experimental.pallas import tpu as pltpu
from jax.experimental.pallas import tpu_sc as plsc

_N = 10000
_E = 320000
_FIN = 128
_H = 256
_HALF = 128
_C = 10
_EPS = 1e-5

_NC = 2
_NS = 16
_CH = 128

_RPT = 624
_RXTRA = _N - _RPT * _NS

_mesh = plsc.VectorSubcoreMesh(
    core_axis_name="c", subcore_axis_name="s", num_cores=_NC, num_subcores=_NS
)


def _zero_vmem_rows(buf, nrows, width):
    def body(i, _):
        for j in range(width // 16):
            buf[i, pl.ds(j * 16, 16)] = jnp.zeros((16,), jnp.float32)
        return 0

    lax.fori_loop(0, nrows, body, 0)


def _fill_ones_rows(buf, nrows, width):
    def body(i, _):
        for j in range(width // 16):
            buf[i, pl.ds(j * 16, 16)] = jnp.ones((16,), jnp.float32)
        return 0

    lax.fori_loop(0, nrows, body, 0)


def _zero_my_shared_slice(acc_sh, zbuf, sid):
    base = sid * _RPT
    for off, size in ((0, 128), (128, 128), (256, 128), (384, 128), (512, 112)):
        pltpu.sync_copy(zbuf.at[pl.ds(0, size)], acc_sh.at[pl.ds(base + off, size)])

    @pl.when(sid == _NS - 1)
    def _():
        pltpu.sync_copy(zbuf.at[pl.ds(0, _RXTRA)],
                        acc_sh.at[pl.ds(_RPT * _NS, _RXTRA)])


def _writeout_my_shared_slice(acc_sh, out_hbm, c, sid):
    base = sid * _RPT
    pltpu.sync_copy(acc_sh.at[pl.ds(base, _RPT)],
                    out_hbm.at[c, pl.ds(base, _RPT)])

    @pl.when(sid == _NS - 1)
    def _():
        pltpu.sync_copy(acc_sh.at[pl.ds(_RPT * _NS, _RXTRA)],
                        out_hbm.at[c, pl.ds(_RPT * _NS, _RXTRA)])


_DW = 16
_DEG_CPW = (_E // _CH) // (_NC * _NS)
_DEG_XTRA = (_E // _CH) - _DEG_CPW * _NC * _NS


def _deg_body(dst_hbm, out_hbm, deg_sh, upd, *rest):
    slots = [rest[3 * i: 3 * i + 3] for i in range(3)]
    c = lax.axis_index("c")
    sid = lax.axis_index("s")

    _zero_vmem_rows(upd, _CH, _DW)
    _zero_my_shared_slice(deg_sh, upd, sid)
    _fill_ones_rows(upd, _CH, _DW)
    plsc.subcore_barrier()

    wid = c * _NS + sid
    ebase = wid * _DEG_CPW * _CH

    def start(s_idx, slot):
        dstb, lsem, _ = slot
        pltpu.async_copy(dst_hbm.at[pl.ds(ebase + s_idx * _CH, _CH)], dstb,
                         lsem)

    def drain(s_idx, slot):
        dstb, lsem, ssem = slot
        pltpu.make_async_copy(dst_hbm.at[pl.ds(ebase + s_idx * _CH, _CH)],
                              dstb, lsem).wait()
        pltpu.async_copy(upd, deg_sh.at[dstb], ssem, add=True)

    def finish(slot):
        dstb, _, ssem = slot
        pltpu.make_async_copy(upd, deg_sh.at[dstb], ssem).wait()

    start(0, slots[0])
    start(1, slots[1])
    drain(0, slots[0])
    start(2, slots[2])
    drain(1, slots[1])

    def ring(k, _):
        for pi in range(3):
            s_idx = 3 * k + 3 + pi
            finish(slots[pi])
            start(s_idx, slots[pi])
            drain(s_idx - 1, slots[(pi + 2) % 3])
        return 0

    lax.fori_loop(0, _DEG_CPW // 3 - 1, ring, 0)
    drain(_DEG_CPW - 1, slots[2])
    for pi in range(3):
        finish(slots[pi])

    @pl.when(wid < _DEG_XTRA)
    def _():
        dstb, lsem, _ = slots[0]
        b = (_DEG_CPW * _NC * _NS + wid) * _CH
        pltpu.sync_copy(dst_hbm.at[pl.ds(b, _CH)], dstb)
        pltpu.sync_copy(upd, deg_sh.at[dstb], add=True)

    plsc.subcore_barrier()
    _writeout_my_shared_slice(deg_sh, out_hbm, c, sid)


_deg_call = pl.kernel(
    _deg_body,
    out_type=jax.ShapeDtypeStruct((_NC, _N, _DW), jnp.float32),
    mesh=_mesh,
    scratch_types=(
        [pltpu.VMEM_SHARED((_N, _DW), jnp.float32),
         pltpu.VMEM((_CH, _DW), jnp.float32)]
        + 3 * [
            pltpu.VMEM((_CH,), jnp.int32),
            pltpu.SemaphoreType.DMA,
            pltpu.SemaphoreType.DMA,
        ]
    ),
)


_NCHUNK = _E // _CH
_CPT = _NCHUNK // _NS
_XTRA = _NCHUNK - _CPT * _NS


def _agg_body(p2_hbm, src_hbm, dst_hbm, out_hbm, s_sh, *rest):
    slots = [rest[6 * i: 6 * i + 6] for i in range(3)]
    c = lax.axis_index("c")
    sid = lax.axis_index("s")

    rows0 = slots[0][3]

    def zrow(i, _):
        for j in range(_HALF // 16):
            rows0[i, pl.ds(j * 16, 16)] = jnp.zeros((16,), jnp.float32)
        return 0

    lax.fori_loop(0, _CH, zrow, 0)
    _zero_my_shared_slice(s_sh, rows0, sid)
    plsc.subcore_barrier()

    ebase = sid * _CPT * _CH

    def start(s_idx, slot):
        srcb, dstb, gi, rows, gsem, _ = slot
        b = ebase + s_idx * _CH
        pltpu.sync_copy(src_hbm.at[pl.ds(b, _CH)], srcb)
        pltpu.sync_copy(dst_hbm.at[pl.ds(b, _CH)], dstb)
        for k in range(_CH // 16):
            sl = pl.ds(k * 16, 16)
            gi[sl] = srcb[sl] * 2 + c
        pltpu.async_copy(p2_hbm.at[gi], rows, gsem)

    def drain(slot):
        _, dstb, gi, rows, gsem, ssem = slot
        pltpu.make_async_copy(p2_hbm.at[gi], rows, gsem).wait()
        pltpu.async_copy(rows, s_sh.at[dstb], ssem, add=True)

    def finish(slot):
        _, dstb, _, rows, _, ssem = slot
        pltpu.make_async_copy(rows, s_sh.at[dstb], ssem).wait()

    start(0, slots[0])
    start(1, slots[1])
    drain(slots[0])
    start(2, slots[2])
    drain(slots[1])

    def ring(k, _):
        for pi in range(3):
            s_idx = 3 * k + 3 + pi
            finish(slots[pi])
            start(s_idx, slots[pi])
            drain(slots[(pi + 2) % 3])
        return 0

    lax.fori_loop(0, _CPT // 3 - 1, ring, 0)
    drain(slots[2])
    for pi in range(3):
        finish(slots[pi])

    @pl.when(sid < _XTRA)
    def _():
        srcb, dstb, gi, rows, gsem, _ = slots[0]
        b = (_CPT * _NS + sid) * _CH
        pltpu.sync_copy(src_hbm.at[pl.ds(b, _CH)], srcb)
        pltpu.sync_copy(dst_hbm.at[pl.ds(b, _CH)], dstb)
        for k in range(_CH // 16):
            sl = pl.ds(k * 16, 16)
            gi[sl] = srcb[sl] * 2 + c
        pltpu.async_copy(p2_hbm.at[gi], rows, gsem).wait()
        pltpu.sync_copy(rows, s_sh.at[dstb], add=True)

    plsc.subcore_barrier()
    _writeout_my_shared_slice(s_sh, out_hbm, c, sid)


_agg_call = pl.kernel(
    _agg_body,
    out_type=jax.ShapeDtypeStruct((_NC, _N, _HALF), jnp.float32),
    mesh=_mesh,
    scratch_types=(
        [pltpu.VMEM_SHARED((_N, _HALF), jnp.float32)]
        + 3 * [
            pltpu.VMEM((_CH,), jnp.int32),
            pltpu.VMEM((_CH,), jnp.int32),
            pltpu.VMEM((_CH,), jnp.int32),
            pltpu.VMEM((_CH, _HALF), jnp.float32),
            pltpu.SemaphoreType.DMA,
            pltpu.SemaphoreType.DMA,
        ]
    ),
)


_BS = 2000
_NG = _N // _BS


def _dinv_block(degp_ref):
    d = degp_ref[0][:, :1] + degp_ref[1][:, :1] + 1.0
    return jax.lax.rsqrt(d)


def _affine(g_ref, b_ref, be_ref):
    gs = g_ref[...][None, :] / jnp.sqrt(jnp.float32(1.0 + _EPS))
    sh = b_ref[...][None, :] * gs + be_ref[...][None, :]
    return gs, sh


def _k0_body(x_ref, w_ref, degp_ref, o_ref):
    dinv = _dinv_block(degp_ref)
    p = (jnp.dot(x_ref[...], w_ref[...], preferred_element_type=jnp.float32)
         * dinv)
    o_ref[...] = p.reshape(2 * _BS, _HALF)


def _mid_body(has_res, *refs):
    if has_res:
        (s_ref, p2_ref, degp_ref, g_ref, b_ref, be_ref, res_ref, w_ref,
         z_ref, pn_ref) = refs
    else:
        (s_ref, p2_ref, degp_ref, g_ref, b_ref, be_ref, w_ref,
         z_ref, pn_ref) = refs
    dinv = _dinv_block(degp_ref)
    gs, sh = _affine(g_ref, b_ref, be_ref)
    p = p2_ref[...].reshape(_BS, _H)
    s_cat = jnp.concatenate([s_ref[0], s_ref[1]], axis=1)
    a = (s_cat + p) * dinv
    z = jnp.maximum(a * gs + sh, 0.0)
    if has_res:
        z = z + res_ref[...]
    z_ref[...] = z
    pn = (jnp.dot(z, w_ref[...], preferred_element_type=jnp.float32) * dinv)
    pn_ref[...] = pn.reshape(2 * _BS, _HALF)


def _fin_body(s_ref, p2_ref, degp_ref, g_ref, b_ref, be_ref, res_ref, wc_ref,
              bc_ref, acc_ref, o_ref):
    i = pl.program_id(0)
    dinv = _dinv_block(degp_ref)
    gs, sh = _affine(g_ref, b_ref, be_ref)
    p = p2_ref[...].reshape(_BS, _H)
    s_cat = jnp.concatenate([s_ref[0], s_ref[1]], axis=1)
    a = (s_cat + p) * dinv
    z = jnp.maximum(a * gs + sh, 0.0) + res_ref[...]
    part = jnp.sum(z, axis=0, keepdims=True)

    @pl.when(i == 0)
    def _():
        acc_ref[...] = part

    @pl.when(i > 0)
    def _():
        acc_ref[...] = acc_ref[...] + part

    @pl.when(i == pl.num_programs(0) - 1)
    def _():
        pooled = acc_ref[...] * (1.0 / _N)
        o_ref[...] = (
            jnp.dot(pooled, wc_ref[...], preferred_element_type=jnp.float32)
            + bc_ref[...][None, :]
        )


def _row_spec(w):
    return pl.BlockSpec((_BS, w), lambda i: (i, 0))


def _fixed_spec(r, w):
    return pl.BlockSpec((r, w), lambda i: (0, 0))


_P2_SPEC = pl.BlockSpec((2 * _BS, _HALF), lambda i: (i, 0))
_S_SPEC = pl.BlockSpec((_NC, _BS, _HALF), lambda i: (0, i, 0))
_DEGP_SPEC = pl.BlockSpec((_NC, _BS, _DW), lambda i: (0, i, 0))
_VEC_SPEC = pl.BlockSpec((_H,), lambda i: (0,))

_P2_TYPE = jax.ShapeDtypeStruct((2 * _N, _HALF), jnp.float32)
_Z_TYPE = jax.ShapeDtypeStruct((_N, _H), jnp.float32)

_k0 = pl.pallas_call(
    _k0_body,
    grid=(_NG,),
    in_specs=[_row_spec(_FIN), _fixed_spec(_FIN, _H), _DEGP_SPEC],
    out_specs=_P2_SPEC,
    out_shape=_P2_TYPE,
)

_k_mid1 = pl.pallas_call(
    functools.partial(_mid_body, False),
    grid=(_NG,),
    in_specs=[_S_SPEC, _P2_SPEC, _DEGP_SPEC, _VEC_SPEC, _VEC_SPEC, _VEC_SPEC,
              _fixed_spec(_H, _H)],
    out_specs=[_row_spec(_H), _P2_SPEC],
    out_shape=[_Z_TYPE, _P2_TYPE],
)

_k_mid2 = pl.pallas_call(
    functools.partial(_mid_body, True),
    grid=(_NG,),
    in_specs=[_S_SPEC, _P2_SPEC, _DEGP_SPEC, _VEC_SPEC, _VEC_SPEC, _VEC_SPEC,
              _row_spec(_H), _fixed_spec(_H, _H)],
    out_specs=[_row_spec(_H), _P2_SPEC],
    out_shape=[_Z_TYPE, _P2_TYPE],
)

_k_fin = pl.pallas_call(
    _fin_body,
    grid=(_NG,),
    in_specs=[_S_SPEC, _P2_SPEC, _DEGP_SPEC, _VEC_SPEC, _VEC_SPEC, _VEC_SPEC,
              _row_spec(_H), _fixed_spec(_H, _C),
              pl.BlockSpec((_C,), lambda i: (0,))],
    out_specs=[_fixed_spec(1, _H), _fixed_spec(1, _C)],
    out_shape=[jax.ShapeDtypeStruct((1, _H), jnp.float32),
               jax.ShapeDtypeStruct((1, _C), jnp.float32)],
)


def kernel(x, edge_index, W0, b0, W1, b1, W2, b2, g0, be0, g1, be1, g2, be2,
           Wc, bc):
    src = edge_index[0]
    dst = edge_index[1]

    degp = _deg_call(dst)
    p2 = _k0(x, W0, degp)
    s = _agg_call(p2, src, dst)
    z1, p2 = _k_mid1(s, p2, degp, g0, b0, be0, W1)
    s = _agg_call(p2, src, dst)
    z2, p2 = _k_mid2(s, p2, degp, g1, b1, be1, z1, W2)
    s = _agg_call(p2, src, dst)
    _, out = _k_fin(s, p2, degp, g2, b2, be2, z2, Wc, bc)
    return out
